# baseline (speedup 1.0000x reference)
import functools

import jax
import jax.numpy as jnp
from jax import lax
from jax.experimental import pallas as pl
from jax.experimental.pallas import tpu as pltpu
from jax.experimental.pallas import tpu_sc as plsc

T = 2048
D = 768
H = 512
E = 16
BLK = 256
NT = (T * 2 + E * BLK) // BLK
PAD_T = NT * BLK
NC, NS = 2, 16
NW = NC * NS
CHUNK = T // NW
SHARED_NT = T // BLK
NT2 = NT + SHARED_NT


HD = D // 2


def _pack_rows(y):
    ua = lax.bitcast_convert_type(y[:, :HD], jnp.uint32)
    ub = lax.bitcast_convert_type(y[:, HD:], jnp.uint32)
    ra = (ua + 0x7FFF + ((ua >> 16) & 1)) >> 16
    rb = (ub + 0x7FFF + ((ub >> 16) & 1)) >> 16
    return lax.bitcast_convert_type((rb << 16) | ra, jnp.float32)


def _unpack_rows(w):
    u = lax.bitcast_convert_type(w, jnp.uint32)
    lo = lax.bitcast_convert_type(u << 16, jnp.float32)
    hi = lax.bitcast_convert_type(u & jnp.uint32(0xFFFF0000), jnp.float32)
    return jnp.concatenate([lo, hi], axis=1)


def _cumsum_rows(a):
    d = 1
    while d < a.shape[0]:
        a = a + jnp.concatenate([jnp.zeros((d, a.shape[1]), a.dtype), a[:-d]], axis=0)
        d *= 2
    return a


def _cumsum_lanes(a):
    d = 1
    while d < a.shape[1]:
        a = a + jnp.concatenate([jnp.zeros((a.shape[0], d), a.dtype), a[:, :-d]], axis=1)
        d *= 2
    return a


def _router_body(xf_ref, wg_ref, eb_ref, pos0_ref, pos1_ref, w0_ref, w1_ref,
                 meta_ref, mv_ref, xq_ref):
    x = xf_ref[...]
    xq_ref[...] = _pack_rows(x)
    wg = wg_ref[...]

    st = lax.dot_general(wg, x, (((1,), (1,)), ((), ())),
                         preferred_element_type=jnp.float32)
    bt = st + eb_ref[...]
    idt = lax.broadcasted_iota(jnp.int32, (E, T), 0)

    scores = lax.dot_general(x, wg, (((1,), (1,)), ((), ())),
                             preferred_element_type=jnp.float32)
    idx = lax.broadcasted_iota(jnp.int32, (T, E), 1)
    v1 = jnp.max(scores, axis=1, keepdims=True)
    iu1 = jnp.min(jnp.where(scores == v1, idx, E), axis=1, keepdims=True)
    v2 = jnp.max(jnp.where(idx == iu1, -jnp.inf, scores), axis=1, keepdims=True)
    s1 = jax.nn.sigmoid(v1)
    s2 = jax.nn.sigmoid(v2)
    w0_ref[...] = jnp.broadcast_to(s1 / (s1 + s2), (T, 16))
    w1_ref[...] = jnp.broadcast_to(s2 / (s1 + s2), (T, 16))
    m1 = jnp.max(bt, axis=0, keepdims=True)
    i1 = jnp.min(jnp.where(bt == m1, idt, E), axis=0, keepdims=True)
    bt2 = jnp.where(idt == i1, -jnp.inf, bt)
    m2 = jnp.max(bt2, axis=0, keepdims=True)
    i2 = jnp.min(jnp.where(bt2 == m2, idt, E), axis=0, keepdims=True)

    oh0 = (idt == i1).astype(jnp.float32)
    oh1 = (idt == i2).astype(jnp.float32)
    c0 = _cumsum_lanes(oh0)
    c1 = _cumsum_lanes(oh1)
    rank0 = jnp.sum((c0 - oh0) * oh0, axis=0, keepdims=True)
    rank1 = jnp.sum((c1 - oh1) * oh1, axis=0, keepdims=True)
    counts0 = jnp.sum(oh0, axis=1, keepdims=True)
    counts1 = jnp.sum(oh1, axis=1, keepdims=True)
    counts = counts0 + counts1
    pc = jnp.ceil(counts * (1.0 / BLK)) * BLK
    gstart = _cumsum_rows(pc) - pc
    base0 = jnp.sum(gstart * oh0, axis=0, keepdims=True)
    base1 = jnp.sum((gstart + counts0) * oh1, axis=0, keepdims=True)
    pos0_ref[...] = (base0 + rank0).astype(jnp.int32).reshape(T)
    pos1_ref[...] = (base1 + rank1).astype(jnp.int32).reshape(T)

    total = jnp.sum(pc, axis=0, keepdims=True)
    ts = (lax.broadcasted_iota(jnp.int32, (1, NT), 1) * BLK).astype(jnp.float32)
    expert = jnp.sum((ts >= gstart).astype(jnp.int32), axis=0,
                     keepdims=True) - 1
    tile_i = lax.broadcasted_iota(jnp.int32, (1, NT), 1)
    active = (ts < total).astype(jnp.int32)
    ntiles = jnp.sum(active, axis=1, keepdims=True)
    clamp = jnp.minimum(tile_i, ntiles - 1)
    last_e = jnp.sum(jnp.where(tile_i == ntiles - 1, expert, 0),
                     axis=1, keepdims=True)
    expert = jnp.where(active == 1, expert, last_e)
    sh_pad = (1, SHARED_NT)
    expert = jnp.concatenate([expert, jnp.broadcast_to(last_e, sh_pad)], axis=1)
    active = jnp.concatenate([active, jnp.zeros(sh_pad, jnp.int32)], axis=1)
    clamp = jnp.concatenate([clamp, jnp.broadcast_to(ntiles - 1, sh_pad)],
                            axis=1)
    meta_ref[...] = jnp.concatenate([expert, active, clamp], axis=0)

    freq = counts * (1.0 / (T * 2))
    fmax = jnp.max(freq, axis=0, keepdims=True)
    fmean = jnp.sum(freq, axis=0, keepdims=True) * (1.0 / E)
    mv_ref[...] = (fmax - fmean) / fmean


def _router(xf, wg, eb):
    return pl.pallas_call(
        _router_body,
        out_shape=(
            jax.ShapeDtypeStruct((T,), jnp.int32),
            jax.ShapeDtypeStruct((T,), jnp.int32),
            jax.ShapeDtypeStruct((T, 16), jnp.float32),
            jax.ShapeDtypeStruct((T, 16), jnp.float32),
            jax.ShapeDtypeStruct((3, NT2), jnp.int32),
            jax.ShapeDtypeStruct((1, 1), jnp.float32),
            jax.ShapeDtypeStruct((T, HD), jnp.float32),
        ),
    )(xf, wg, eb)


def _dispatch_body(xq_hbm, pos0_hbm, pos1_hbm, xs_hbm, rows_v, idx0_v, idx1_v,
                   sem0, sem1):
    wid = lax.axis_index("s") * NC + lax.axis_index("c")
    base = wid * CHUNK
    pltpu.sync_copy(xq_hbm.at[pl.ds(base, CHUNK)], rows_v)
    pltpu.sync_copy(pos0_hbm.at[pl.ds(base, CHUNK)], idx0_v)
    pltpu.sync_copy(pos1_hbm.at[pl.ds(base, CHUNK)], idx1_v)
    c0 = pltpu.async_copy(rows_v, xs_hbm.at[idx0_v], sem0)
    c1 = pltpu.async_copy(rows_v, xs_hbm.at[idx1_v], sem1)
    c0.wait()
    c1.wait()


def _dispatch(xq, pos0, pos1):
    mesh = plsc.VectorSubcoreMesh(core_axis_name="c", subcore_axis_name="s",
                                  num_cores=NC, num_subcores=NS)
    return pl.kernel(
        _dispatch_body,
        out_type=jax.ShapeDtypeStruct((PAD_T, HD), jnp.float32),
        mesh=mesh,
        scratch_types=[
            pltpu.VMEM((CHUNK, HD), jnp.float32),
            pltpu.VMEM((CHUNK,), jnp.int32),
            pltpu.VMEM((CHUNK,), jnp.int32),
            pltpu.SemaphoreType.DMA,
            pltpu.SemaphoreType.DMA,
        ],
    )(xq, pos0, pos1)


def _swiglu(x, w1, w2, w3):
    a = lax.dot_general(x, w1.astype(jnp.bfloat16), (((1,), (1,)), ((), ())),
                        preferred_element_type=jnp.float32)
    b = lax.dot_general(x, w3.astype(jnp.bfloat16), (((1,), (1,)), ((), ())),
                        preferred_element_type=jnp.float32)
    h = (jax.nn.silu(a) * b).astype(jnp.bfloat16)
    return lax.dot_general(h, w2.astype(jnp.bfloat16), (((1,), (1,)), ((), ())),
                           preferred_element_type=jnp.float32)


def _expert_body(meta_ref, xs_ref, xq_ref, w1_ref, w2_ref, w3_ref,
                 sw1_ref, sw2_ref, sw3_ref, ys_ref, shq_ref):
    i = pl.program_id(0)

    @pl.when(meta_ref[1, i] == 1)
    def _():
        x = _unpack_rows(xs_ref[...]).astype(jnp.bfloat16)
        ys_ref[...] = _pack_rows(_swiglu(x, w1_ref[0], w2_ref[0], w3_ref[0]))

    @pl.when(i >= NT)
    def _():
        x = _unpack_rows(xq_ref[...]).astype(jnp.bfloat16)
        shq_ref[...] = _pack_rows(
            _swiglu(x, sw1_ref[...], sw2_ref[...], sw3_ref[...]))


def _expert_mlp(meta, xs, xq, w1, w2, w3, sw1, sw2, sw3):
    grid_spec = pltpu.PrefetchScalarGridSpec(
        num_scalar_prefetch=1,
        grid=(NT2,),
        in_specs=[
            pl.BlockSpec((BLK, HD), lambda i, m: (m[2, i], 0)),
            pl.BlockSpec((BLK, HD), lambda i, m: (jnp.maximum(i - NT, 0), 0)),
            pl.BlockSpec((1, H, D), lambda i, m: (m[0, i], 0, 0)),
            pl.BlockSpec((1, D, H), lambda i, m: (m[0, i], 0, 0)),
            pl.BlockSpec((1, H, D), lambda i, m: (m[0, i], 0, 0)),
            pl.BlockSpec((H, D), lambda i, m: (0, 0)),
            pl.BlockSpec((D, H), lambda i, m: (0, 0)),
            pl.BlockSpec((H, D), lambda i, m: (0, 0)),
        ],
        out_specs=[
            pl.BlockSpec((BLK, HD), lambda i, m: (m[2, i], 0)),
            pl.BlockSpec((BLK, HD), lambda i, m: (jnp.maximum(i - NT, 0), 0)),
        ],
    )
    return pl.pallas_call(
        _expert_body,
        grid_spec=grid_spec,
        out_shape=(
            jax.ShapeDtypeStruct((PAD_T, HD), jnp.float32),
            jax.ShapeDtypeStruct((T, HD), jnp.float32),
        ),
        compiler_params=pltpu.CompilerParams(
            dimension_semantics=("arbitrary",)),
    )(meta, xs, xq, w1, w2, w3, sw1, sw2, sw3)


def _combine_body(ys_hbm, shared_hbm, pos0_hbm, pos1_hbm, w0_hbm, w1_hbm,
                  out_hbm, out_v, sh_v, g0_v, g1_v, idx0_v, idx1_v, w0_v, w1_v,
                  sem0, sem1):
    wid = lax.axis_index("s") * NC + lax.axis_index("c")
    base = wid * CHUNK
    pltpu.sync_copy(pos0_hbm.at[pl.ds(base, CHUNK)], idx0_v)
    pltpu.sync_copy(pos1_hbm.at[pl.ds(base, CHUNK)], idx1_v)
    c0 = pltpu.async_copy(ys_hbm.at[idx0_v], g0_v, sem0)
    c1 = pltpu.async_copy(ys_hbm.at[idx1_v], g1_v, sem1)
    pltpu.sync_copy(shared_hbm.at[pl.ds(base, CHUNK)], sh_v)
    pltpu.sync_copy(w0_hbm.at[pl.ds(base, CHUNK)], w0_v)
    pltpu.sync_copy(w1_hbm.at[pl.ds(base, CHUNK)], w1_v)
    c0.wait()
    c1.wait()

    hi_mask = jnp.full((16,), 0xFFFF0000, jnp.uint32)

    def unpk(u):
        return (plsc.bitcast(u << 16, jnp.float32),
                plsc.bitcast(u & hi_mask, jnp.float32))

    for half in range(2):
        hbase = half * (CHUNK // 2)

        @plsc.parallel_loop(0, CHUNK // 2)
        def row(i, hbase=hbase):
            r = hbase + i
            a = w0_v[r, :]
            b = w1_v[r, :]
            for j in range(HD // 16):
                col = j * 16
                lo0, hi0 = unpk(plsc.bitcast(g0_v[r, pl.ds(col, 16)], jnp.uint32))
                lo1, hi1 = unpk(plsc.bitcast(g1_v[r, pl.ds(col, 16)], jnp.uint32))
                los, his = unpk(plsc.bitcast(sh_v[r, pl.ds(col, 16)], jnp.uint32))
                out_v[i, pl.ds(col, 16)] = los + a * lo0 + b * lo1
                out_v[i, pl.ds(HD + col, 16)] = his + a * hi0 + b * hi1

        pltpu.sync_copy(out_v, out_hbm.at[pl.ds(base + hbase, CHUNK // 2)])


def _combine(ys, shared, pos0, pos1, w0, w1):
    mesh = plsc.VectorSubcoreMesh(core_axis_name="c", subcore_axis_name="s",
                                  num_cores=NC, num_subcores=NS)
    return pl.kernel(
        _combine_body,
        out_type=jax.ShapeDtypeStruct((T, D), jnp.float32),
        mesh=mesh,
        compiler_params=pltpu.CompilerParams(needs_layout_passes=False),
        scratch_types=[
            pltpu.VMEM((CHUNK // 2, D), jnp.float32),
            pltpu.VMEM((CHUNK, HD), jnp.float32),
            pltpu.VMEM((CHUNK, HD), jnp.float32),
            pltpu.VMEM((CHUNK, HD), jnp.float32),
            pltpu.VMEM((CHUNK,), jnp.int32),
            pltpu.VMEM((CHUNK,), jnp.int32),
            pltpu.VMEM((CHUNK, 16), jnp.float32),
            pltpu.VMEM((CHUNK, 16), jnp.float32),
            pltpu.SemaphoreType.DMA,
            pltpu.SemaphoreType.DMA,
        ],
    )(ys, shared, pos0, pos1, w0, w1)


def kernel(x, Wg, W1, W2, W3, SW1, SW2, SW3, e_bias):
    xf = x.reshape(T, D)
    pos0, pos1, w0, w1, meta, mv, xq = _router(xf, Wg, e_bias.reshape(E, 1))
    xs = _dispatch(xq, pos0, pos1)
    ys, shared = _expert_mlp(meta, xs, xq, W1, W2, W3, SW1[0], SW2[0], SW3[0])
    out = _combine(ys, shared, pos0, pos1, w0, w1)
    return out.reshape(1, T, D), jnp.float32(0.0), mv.reshape(())


# speedup vs baseline: 1.0695x; 1.0695x over previous
import functools

import jax
import jax.numpy as jnp
from jax import lax
from jax.experimental import pallas as pl
from jax.experimental.pallas import tpu as pltpu
from jax.experimental.pallas import tpu_sc as plsc

T = 2048
D = 768
H = 512
E = 16
BLK = 256
NT = (T * 2 + E * BLK) // BLK
PAD_T = NT * BLK
NC, NS = 2, 16
NW = NC * NS
CHUNK = T // NW


HD = D // 2


def _pack_rows(y):
    ua = lax.bitcast_convert_type(y[:, :HD], jnp.uint32)
    ub = lax.bitcast_convert_type(y[:, HD:], jnp.uint32)
    ra = (ua + 0x7FFF + ((ua >> 16) & 1)) >> 16
    rb = (ub + 0x7FFF + ((ub >> 16) & 1)) >> 16
    return lax.bitcast_convert_type((rb << 16) | ra, jnp.float32)


def _unpack_rows(w):
    u = lax.bitcast_convert_type(w, jnp.uint32)
    lo = lax.bitcast_convert_type(u << 16, jnp.float32)
    hi = lax.bitcast_convert_type(u & jnp.uint32(0xFFFF0000), jnp.float32)
    return jnp.concatenate([lo, hi], axis=1)


def _cumsum_rows(a):
    d = 1
    while d < a.shape[0]:
        a = a + jnp.concatenate([jnp.zeros((d, a.shape[1]), a.dtype), a[:-d]], axis=0)
        d *= 2
    return a


def _cumsum_lanes(a):
    d = 1
    while d < a.shape[1]:
        a = a + jnp.concatenate([jnp.zeros((a.shape[0], d), a.dtype), a[:, :-d]], axis=1)
        d *= 2
    return a


def _router_body(xf_ref, wg_ref, eb_ref, pos0_ref, pos1_ref, w0_ref, w1_ref,
                 meta_ref, mv_ref, xq_ref):
    x = xf_ref[...]
    xq_ref[...] = _pack_rows(x)
    wg = wg_ref[...]

    st = lax.dot_general(wg, x, (((1,), (1,)), ((), ())),
                         preferred_element_type=jnp.float32)
    bt = st + eb_ref[...]
    idt = lax.broadcasted_iota(jnp.int32, (E, T), 0)

    scores = lax.dot_general(x, wg, (((1,), (1,)), ((), ())),
                             preferred_element_type=jnp.float32)
    idx = lax.broadcasted_iota(jnp.int32, (T, E), 1)
    v1 = jnp.max(scores, axis=1, keepdims=True)
    iu1 = jnp.min(jnp.where(scores == v1, idx, E), axis=1, keepdims=True)
    v2 = jnp.max(jnp.where(idx == iu1, -jnp.inf, scores), axis=1, keepdims=True)
    s1 = jax.nn.sigmoid(v1)
    s2 = jax.nn.sigmoid(v2)
    w0_ref[...] = jnp.broadcast_to(s1 / (s1 + s2), (T, 16))
    w1_ref[...] = jnp.broadcast_to(s2 / (s1 + s2), (T, 16))
    m1 = jnp.max(bt, axis=0, keepdims=True)
    i1 = jnp.min(jnp.where(bt == m1, idt, E), axis=0, keepdims=True)
    bt2 = jnp.where(idt == i1, -jnp.inf, bt)
    m2 = jnp.max(bt2, axis=0, keepdims=True)
    i2 = jnp.min(jnp.where(bt2 == m2, idt, E), axis=0, keepdims=True)

    oh0 = (idt == i1).astype(jnp.float32)
    oh1 = (idt == i2).astype(jnp.float32)
    c0 = _cumsum_lanes(oh0)
    c1 = _cumsum_lanes(oh1)
    rank0 = jnp.sum((c0 - oh0) * oh0, axis=0, keepdims=True)
    rank1 = jnp.sum((c1 - oh1) * oh1, axis=0, keepdims=True)
    counts0 = jnp.sum(oh0, axis=1, keepdims=True)
    counts1 = jnp.sum(oh1, axis=1, keepdims=True)
    counts = counts0 + counts1
    pc = jnp.ceil(counts * (1.0 / BLK)) * BLK
    gstart = _cumsum_rows(pc) - pc
    base0 = jnp.sum(gstart * oh0, axis=0, keepdims=True)
    base1 = jnp.sum((gstart + counts0) * oh1, axis=0, keepdims=True)
    pos0_ref[...] = (base0 + rank0).astype(jnp.int32).reshape(T)
    pos1_ref[...] = (base1 + rank1).astype(jnp.int32).reshape(T)

    total = jnp.sum(pc, axis=0, keepdims=True)
    ts = (lax.broadcasted_iota(jnp.int32, (1, NT), 1) * BLK).astype(jnp.float32)
    expert = jnp.sum((ts >= gstart).astype(jnp.int32), axis=0,
                     keepdims=True) - 1
    tile_i = lax.broadcasted_iota(jnp.int32, (1, NT), 1)
    active = (ts < total).astype(jnp.int32)
    ntiles = jnp.sum(active, axis=1, keepdims=True)
    clamp = jnp.minimum(tile_i, ntiles - 1)
    last_e = jnp.sum(jnp.where(tile_i == ntiles - 1, expert, 0),
                     axis=1, keepdims=True)
    expert = jnp.where(active == 1, expert, last_e)
    meta_ref[...] = jnp.concatenate([expert, active, clamp], axis=0)

    freq = counts * (1.0 / (T * 2))
    fmax = jnp.max(freq, axis=0, keepdims=True)
    fmean = jnp.sum(freq, axis=0, keepdims=True) * (1.0 / E)
    mv_ref[...] = (fmax - fmean) / fmean


def _router(xf, wg, eb):
    return pl.pallas_call(
        _router_body,
        out_shape=(
            jax.ShapeDtypeStruct((T,), jnp.int32),
            jax.ShapeDtypeStruct((T,), jnp.int32),
            jax.ShapeDtypeStruct((T, 16), jnp.float32),
            jax.ShapeDtypeStruct((T, 16), jnp.float32),
            jax.ShapeDtypeStruct((3, NT), jnp.int32),
            jax.ShapeDtypeStruct((1, 1), jnp.float32),
            jax.ShapeDtypeStruct((T, HD), jnp.float32),
        ),
    )(xf, wg, eb)


def _dispatch_body(xq_hbm, pos0_hbm, pos1_hbm, xs_hbm, rows_v, idx0_v, idx1_v,
                   sem0, sem1):
    wid = lax.axis_index("s") * NC + lax.axis_index("c")
    base = wid * CHUNK
    pltpu.sync_copy(xq_hbm.at[pl.ds(base, CHUNK)], rows_v)
    pltpu.sync_copy(pos0_hbm.at[pl.ds(base, CHUNK)], idx0_v)
    pltpu.sync_copy(pos1_hbm.at[pl.ds(base, CHUNK)], idx1_v)
    c0 = pltpu.async_copy(rows_v, xs_hbm.at[idx0_v], sem0)
    c1 = pltpu.async_copy(rows_v, xs_hbm.at[idx1_v], sem1)
    c0.wait()
    c1.wait()


def _dispatch(xq, pos0, pos1):
    mesh = plsc.VectorSubcoreMesh(core_axis_name="c", subcore_axis_name="s",
                                  num_cores=NC, num_subcores=NS)
    return pl.kernel(
        _dispatch_body,
        out_type=jax.ShapeDtypeStruct((PAD_T, HD), jnp.float32),
        mesh=mesh,
        scratch_types=[
            pltpu.VMEM((CHUNK, HD), jnp.float32),
            pltpu.VMEM((CHUNK,), jnp.int32),
            pltpu.VMEM((CHUNK,), jnp.int32),
            pltpu.SemaphoreType.DMA,
            pltpu.SemaphoreType.DMA,
        ],
    )(xq, pos0, pos1)


def _swiglu(x, w1, w2, w3):
    a = lax.dot_general(x, w1.astype(jnp.bfloat16), (((1,), (1,)), ((), ())),
                        preferred_element_type=jnp.float32)
    b = lax.dot_general(x, w3.astype(jnp.bfloat16), (((1,), (1,)), ((), ())),
                        preferred_element_type=jnp.float32)
    h = (jax.nn.silu(a) * b).astype(jnp.bfloat16)
    return lax.dot_general(h, w2.astype(jnp.bfloat16), (((1,), (1,)), ((), ())),
                           preferred_element_type=jnp.float32)


def _expert_body(meta_ref, xs_ref, w1_ref, w2_ref, w3_ref, ys_ref):
    i = pl.program_id(0)

    @pl.when(meta_ref[1, i] == 1)
    def _():
        x = _unpack_rows(xs_ref[...]).astype(jnp.bfloat16)
        ys_ref[...] = _pack_rows(_swiglu(x, w1_ref[0], w2_ref[0], w3_ref[0]))


def _expert_mlp(meta, xs, w1, w2, w3):
    grid_spec = pltpu.PrefetchScalarGridSpec(
        num_scalar_prefetch=1,
        grid=(NT,),
        in_specs=[
            pl.BlockSpec((BLK, HD), lambda i, m: (m[2, i], 0)),
            pl.BlockSpec((1, H, D), lambda i, m: (m[0, i], 0, 0)),
            pl.BlockSpec((1, D, H), lambda i, m: (m[0, i], 0, 0)),
            pl.BlockSpec((1, H, D), lambda i, m: (m[0, i], 0, 0)),
        ],
        out_specs=pl.BlockSpec((BLK, HD), lambda i, m: (m[2, i], 0)),
    )
    return pl.pallas_call(
        _expert_body,
        grid_spec=grid_spec,
        out_shape=jax.ShapeDtypeStruct((PAD_T, HD), jnp.float32),
        compiler_params=pltpu.CompilerParams(
            dimension_semantics=("arbitrary",)),
    )(meta, xs, w1, w2, w3)


def _shared_body(xq_ref, sw1_ref, sw2_ref, sw3_ref, out_ref):
    x = _unpack_rows(xq_ref[...]).astype(jnp.bfloat16)
    out_ref[...] = _pack_rows(_swiglu(x, sw1_ref[...], sw2_ref[...], sw3_ref[...]))


def _shared_mlp(xq, sw1, sw2, sw3):
    sblk = 512
    nblk = T // sblk
    return pl.pallas_call(
        _shared_body,
        grid=(nblk,),
        in_specs=[
            pl.BlockSpec((sblk, HD), lambda i: (i, 0)),
            pl.BlockSpec((H, D), lambda i: (0, 0)),
            pl.BlockSpec((D, H), lambda i: (0, 0)),
            pl.BlockSpec((H, D), lambda i: (0, 0)),
        ],
        out_specs=pl.BlockSpec((sblk, HD), lambda i: (i, 0)),
        out_shape=jax.ShapeDtypeStruct((T, HD), jnp.float32),
        compiler_params=pltpu.CompilerParams(
            dimension_semantics=("arbitrary",)),
    )(xq, sw1, sw2, sw3)


def _combine_body(ys_hbm, shared_hbm, pos0_hbm, pos1_hbm, w0_hbm, w1_hbm,
                  out_hbm, out_v, sh_v, g0_v, g1_v, idx0_v, idx1_v, w0_v, w1_v,
                  sem0, sem1):
    wid = lax.axis_index("s") * NC + lax.axis_index("c")
    base = wid * CHUNK
    pltpu.sync_copy(pos0_hbm.at[pl.ds(base, CHUNK)], idx0_v)
    pltpu.sync_copy(pos1_hbm.at[pl.ds(base, CHUNK)], idx1_v)
    c0 = pltpu.async_copy(ys_hbm.at[idx0_v], g0_v, sem0)
    c1 = pltpu.async_copy(ys_hbm.at[idx1_v], g1_v, sem1)
    pltpu.sync_copy(shared_hbm.at[pl.ds(base, CHUNK)], sh_v)
    pltpu.sync_copy(w0_hbm.at[pl.ds(base, CHUNK)], w0_v)
    pltpu.sync_copy(w1_hbm.at[pl.ds(base, CHUNK)], w1_v)
    c0.wait()
    c1.wait()

    hi_mask = jnp.full((16,), 0xFFFF0000, jnp.uint32)

    def unpk(u):
        return (plsc.bitcast(u << 16, jnp.float32),
                plsc.bitcast(u & hi_mask, jnp.float32))

    for half in range(2):
        hbase = half * (CHUNK // 2)

        @plsc.parallel_loop(0, CHUNK // 2)
        def row(i, hbase=hbase):
            r = hbase + i
            a = w0_v[r, :]
            b = w1_v[r, :]
            for j in range(HD // 16):
                col = j * 16
                lo0, hi0 = unpk(plsc.bitcast(g0_v[r, pl.ds(col, 16)], jnp.uint32))
                lo1, hi1 = unpk(plsc.bitcast(g1_v[r, pl.ds(col, 16)], jnp.uint32))
                los, his = unpk(plsc.bitcast(sh_v[r, pl.ds(col, 16)], jnp.uint32))
                out_v[i, pl.ds(col, 16)] = los + a * lo0 + b * lo1
                out_v[i, pl.ds(HD + col, 16)] = his + a * hi0 + b * hi1

        pltpu.sync_copy(out_v, out_hbm.at[pl.ds(base + hbase, CHUNK // 2)])


def _combine(ys, shared, pos0, pos1, w0, w1):
    mesh = plsc.VectorSubcoreMesh(core_axis_name="c", subcore_axis_name="s",
                                  num_cores=NC, num_subcores=NS)
    return pl.kernel(
        _combine_body,
        out_type=jax.ShapeDtypeStruct((T, D), jnp.float32),
        mesh=mesh,
        compiler_params=pltpu.CompilerParams(needs_layout_passes=False),
        scratch_types=[
            pltpu.VMEM((CHUNK // 2, D), jnp.float32),
            pltpu.VMEM((CHUNK, HD), jnp.float32),
            pltpu.VMEM((CHUNK, HD), jnp.float32),
            pltpu.VMEM((CHUNK, HD), jnp.float32),
            pltpu.VMEM((CHUNK,), jnp.int32),
            pltpu.VMEM((CHUNK,), jnp.int32),
            pltpu.VMEM((CHUNK, 16), jnp.float32),
            pltpu.VMEM((CHUNK, 16), jnp.float32),
            pltpu.SemaphoreType.DMA,
            pltpu.SemaphoreType.DMA,
        ],
    )(ys, shared, pos0, pos1, w0, w1)


def kernel(x, Wg, W1, W2, W3, SW1, SW2, SW3, e_bias):
    xf = x.reshape(T, D)
    pos0, pos1, w0, w1, meta, mv, xq = _router(xf, Wg, e_bias.reshape(E, 1))
    shared = _shared_mlp(xq, SW1[0], SW2[0], SW3[0])
    xs = _dispatch(xq, pos0, pos1)
    ys = _expert_mlp(meta, xs, W1, W2, W3)
    out = _combine(ys, shared, pos0, pos1, w0, w1)
    return out.reshape(1, T, D), jnp.float32(0.0), mv.reshape(())
